# baseline (speedup 1.0000x reference)
import functools

import jax
import jax.numpy as jnp
from jax import lax
from jax.experimental import pallas as pl
from jax.experimental.pallas import tpu as pltpu
from jax.experimental.pallas import tpu_sc as plsc

_NC = 2
_NS = 16
_NW = _NC * _NS
_L = 16

_CHUNK = 16384


def _take(tab, idx):
    return tab.at[idx].get(mode="promise_in_bounds")


@functools.lru_cache(maxsize=None)
def _build_sc_kernel(n: int, k: int):
    assert k == 8, "kernel is specialized to K == 8 mixture components"
    per_w = n // _NW
    assert per_w * _NW == n
    chunk = min(_CHUNK, per_w)
    n_ch = per_w // chunk
    assert n_ch * chunk == per_w
    km1 = k - 1

    mesh = plsc.VectorSubcoreMesh(
        core_axis_name="c", subcore_axis_name="s", num_cores=_NC, num_subcores=_NS
    )

    @functools.partial(
        pl.kernel,
        mesh=mesh,
        out_type=jax.ShapeDtypeStruct((n,), jnp.float32),
        scratch_types=[
            pltpu.VMEM((chunk,), jnp.int32),
            pltpu.VMEM((chunk,), jnp.float32),
            pltpu.VMEM((2 * k,), jnp.float32),
            pltpu.SemaphoreType.DMA,
        ],
    )
    def sc_kernel(eps_hbm, ms_hbm, tab_hbm, out_hbm, idx_v, g_v, tab_v, sem):
        wid = lax.axis_index("s") * _NC + lax.axis_index("c")
        base = wid * per_w

        pltpu.sync_copy(tab_hbm, tab_v)
        tab = tab_v[...]

        iota_k = lax.iota(jnp.int32, _L) * k

        def chunk_body(ch, carry):
            off = base + ch * chunk
            pltpu.sync_copy(ms_hbm.at[pl.ds(off, chunk)], idx_v)

            off_k = off * k

            def p1(j, c):
                sl = pl.ds(j * _L, _L)
                m = idx_v[sl]
                idx_v[sl] = m + (off_k + j * (_L * k)) + iota_k
                return c

            lax.fori_loop(0, chunk // _L, p1, 0)

            pltpu.async_copy(eps_hbm.at[idx_v], g_v, sem).wait()

            def p2(j, c):
                sl = pl.ds(j * _L, _L)
                fl = idx_v[sl]
                g = g_v[sl]
                m = jnp.bitwise_and(fl, km1)
                lo = _take(tab, m)
                sc = _take(tab, m + k)
                g_v[sl] = lo + sc * g
                return c

            lax.fori_loop(0, chunk // _L, p2, 0)

            pltpu.sync_copy(g_v, out_hbm.at[pl.ds(off, chunk)])
            return carry

        lax.fori_loop(0, n_ch, chunk_body, 0)

    return sc_kernel


def kernel(eps, ms, location, scale):
    n, k = eps.shape
    sc_kernel = _build_sc_kernel(n, k)
    tab = jnp.concatenate(
        [location.astype(jnp.float32), scale.astype(jnp.float32)]
    )
    return sc_kernel(eps.reshape(n * k), ms.astype(jnp.int32), tab)


# speedup vs baseline: 1.0407x; 1.0407x over previous
import functools

import jax
import jax.numpy as jnp
from jax import lax
from jax.experimental import pallas as pl
from jax.experimental.pallas import tpu as pltpu
from jax.experimental.pallas import tpu_sc as plsc

_NC = 2
_NS = 16
_NW = _NC * _NS
_L = 16

_CHUNK = 16384


def _take(tab, idx):
    return tab.at[idx].get(mode="promise_in_bounds")


@functools.lru_cache(maxsize=None)
def _build_sc_kernel(n: int, k: int):
    assert k == 8, "kernel is specialized to K == 8 mixture components"
    per_w = n // _NW
    assert per_w * _NW == n
    chunk = min(_CHUNK, per_w)
    n_ch = per_w // chunk
    assert n_ch * chunk == per_w
    km1 = k - 1

    mesh = plsc.VectorSubcoreMesh(
        core_axis_name="c", subcore_axis_name="s", num_cores=_NC, num_subcores=_NS
    )

    @functools.partial(
        pl.kernel,
        mesh=mesh,
        out_type=jax.ShapeDtypeStruct((n,), jnp.float32),
        scratch_types=[
            pltpu.VMEM((chunk,), jnp.int32),
            pltpu.VMEM((chunk,), jnp.float32),
            pltpu.VMEM((2 * k,), jnp.float32),
            pltpu.SemaphoreType.DMA,
        ],
    )
    def sc_kernel(eps_hbm, ms_hbm, tab_hbm, out_hbm, idx_v, g_v, tab_v, sem):
        wid = lax.axis_index("s") * _NC + lax.axis_index("c")
        base = wid * per_w

        pltpu.sync_copy(tab_hbm, tab_v)
        tab = tab_v[...]

        iota_k = lax.iota(jnp.int32, _L) * k

        def chunk_body(ch, carry):
            off = base + ch * chunk
            pltpu.sync_copy(ms_hbm.at[pl.ds(off, chunk)], idx_v)

            off_k = off * k

            @plsc.parallel_loop(0, chunk, _L, unroll=8)
            def p1(j):
                sl = pl.ds(j, _L)
                idx_v[sl] = idx_v[sl] + (off_k + j * k) + iota_k

            pltpu.async_copy(eps_hbm.at[idx_v], g_v, sem).wait()

            @plsc.parallel_loop(0, chunk, _L, unroll=8)
            def p2(j):
                sl = pl.ds(j, _L)
                fl = idx_v[sl]
                g = g_v[sl]
                m = jnp.bitwise_and(fl, km1)
                lo = _take(tab, m)
                sc = _take(tab, m + k)
                g_v[sl] = lo + sc * g

            pltpu.sync_copy(g_v, out_hbm.at[pl.ds(off, chunk)])
            return carry

        lax.fori_loop(0, n_ch, chunk_body, 0)

    return sc_kernel


def kernel(eps, ms, location, scale):
    n, k = eps.shape
    sc_kernel = _build_sc_kernel(n, k)
    tab = jnp.concatenate(
        [location.astype(jnp.float32), scale.astype(jnp.float32)]
    )
    return sc_kernel(eps.reshape(n * k), ms.astype(jnp.int32), tab)


# speedup vs baseline: 8.7445x; 8.4027x over previous
import functools

import jax
import jax.numpy as jnp
from jax import lax
from jax.experimental import pallas as pl
from jax.experimental.pallas import tpu as pltpu
from jax.experimental.pallas import tpu_sc as plsc

_NC = 2
_NS = 16
_NW = _NC * _NS
_L = 16
_LANES = 128

_CHUNK = 16384


def _take(tab, idx):
    return tab.at[idx].get(mode="promise_in_bounds")


@functools.lru_cache(maxsize=None)
def _build_sc_kernel(n: int, k: int):
    assert k == 8, "kernel is specialized to K == 8 mixture components"
    per_w = n // _NW
    assert per_w * _NW == n
    chunk = min(_CHUNK, per_w)
    n_ch = per_w // chunk
    assert n_ch * chunk == per_w
    assert chunk % _LANES == 0 and n % _LANES == 0
    tile = k * _LANES

    mesh = plsc.VectorSubcoreMesh(
        core_axis_name="c", subcore_axis_name="s", num_cores=_NC, num_subcores=_NS
    )

    @functools.partial(
        pl.kernel,
        mesh=mesh,
        out_type=jax.ShapeDtypeStruct((n,), jnp.float32),
        scratch_types=[
            pltpu.VMEM((chunk,), jnp.int32),
            pltpu.VMEM((chunk,), jnp.float32),
            pltpu.VMEM((2 * k,), jnp.float32),
            pltpu.SemaphoreType.DMA,
        ],
    )
    def sc_kernel(eps_hbm, ms_hbm, tab_hbm, out_hbm, idx_v, g_v, tab_v, sem):
        wid = lax.axis_index("s") * _NC + lax.axis_index("c")
        base = wid * per_w

        pltpu.sync_copy(tab_hbm, tab_v)
        tab = tab_v[...]

        iota = lax.iota(jnp.int32, _L)

        def chunk_body(ch, carry):
            off = base + ch * chunk
            pltpu.sync_copy(ms_hbm.at[pl.ds(off, chunk)], idx_v)

            @plsc.parallel_loop(0, chunk, _L, unroll=8)
            def p1(j):
                sl = pl.ds(j, _L)
                i0 = off + j
                s = (i0 // _LANES) * tile + (i0 % _LANES)
                idx_v[sl] = lax.shift_left(idx_v[sl], 7) + (s + iota)

            pltpu.async_copy(eps_hbm.at[idx_v], g_v, sem).wait()

            @plsc.parallel_loop(0, chunk, _L, unroll=8)
            def p2(j):
                sl = pl.ds(j, _L)
                fl = idx_v[sl]
                g = g_v[sl]
                m = jnp.bitwise_and(lax.shift_right_logical(fl, 7), k - 1)
                lo = _take(tab, m)
                sc = _take(tab, m + k)
                g_v[sl] = lo + sc * g

            pltpu.sync_copy(g_v, out_hbm.at[pl.ds(off, chunk)])
            return carry

        lax.fori_loop(0, n_ch, chunk_body, 0)

    return sc_kernel


def kernel(eps, ms, location, scale):
    n, k = eps.shape
    sc_kernel = _build_sc_kernel(n, k)
    eps_native = (
        eps.reshape(n // _LANES, _LANES, k).transpose(0, 2, 1).reshape(n * k)
    )
    tab = jnp.concatenate(
        [location.astype(jnp.float32), scale.astype(jnp.float32)]
    )
    return sc_kernel(eps_native, ms.astype(jnp.int32), tab)


# speedup vs baseline: 10.0706x; 1.1517x over previous
import functools

import jax
import jax.numpy as jnp
from jax import lax
from jax.experimental import pallas as pl
from jax.experimental.pallas import tpu as pltpu
from jax.experimental.pallas import tpu_sc as plsc

_NC = 2
_NS = 16
_NW = _NC * _NS
_L = 16
_LANES = 128

_CHUNK = 16384


def _take(tab, idx):
    return tab.at[idx].get(mode="promise_in_bounds")


@functools.lru_cache(maxsize=None)
def _build_sc_kernel(n: int, k: int):
    assert k == 8, "kernel is specialized to K == 8 mixture components"
    per_w = n // _NW
    assert per_w * _NW == n
    chunk = min(_CHUNK, per_w)
    n_ch = per_w // chunk
    assert n_ch * chunk == per_w
    assert chunk % _LANES == 0 and n % _LANES == 0
    tile = k * _LANES

    mesh = plsc.VectorSubcoreMesh(
        core_axis_name="c", subcore_axis_name="s", num_cores=_NC, num_subcores=_NS
    )

    @functools.partial(
        pl.kernel,
        mesh=mesh,
        out_type=jax.ShapeDtypeStruct((n,), jnp.float32),
        scratch_types=[
            pltpu.VMEM((chunk,), jnp.int32),
            pltpu.VMEM((chunk,), jnp.int32),
            pltpu.VMEM((chunk,), jnp.float32),
            pltpu.VMEM((chunk,), jnp.float32),
            pltpu.VMEM((2 * k,), jnp.float32),
            pltpu.SemaphoreType.DMA,
            pltpu.SemaphoreType.DMA,
            pltpu.SemaphoreType.DMA,
            pltpu.SemaphoreType.DMA,
        ],
    )
    def sc_kernel(eps_hbm, ms_hbm, tab_hbm, out_hbm,
                  idx0, idx1, g0, g1, tab_v, gs0, gs1, ss0, ss1):
        idx_v = (idx0, idx1)
        g_v = (g0, g1)
        gsem = (gs0, gs1)
        ssem = (ss0, ss1)

        wid = lax.axis_index("s") * _NC + lax.axis_index("c")
        base = wid * per_w

        pltpu.sync_copy(tab_hbm, tab_v)
        tab = tab_v[...]

        iota = lax.iota(jnp.int32, _L)

        def load_and_index(c, b):
            off = base + c * chunk
            pltpu.sync_copy(ms_hbm.at[pl.ds(off, chunk)], idx_v[b])

            @plsc.parallel_loop(0, chunk, _L, unroll=8)
            def p1(j):
                sl = pl.ds(j, _L)
                i0 = off + j
                s = (i0 // _LANES) * tile + (i0 % _LANES)
                idx_v[b][sl] = lax.shift_left(idx_v[b][sl], 7) + (s + iota)

        def start_gather(b):
            return pltpu.async_copy(eps_hbm.at[idx_v[b]], g_v[b], gsem[b])

        def transform(b):
            @plsc.parallel_loop(0, chunk, _L, unroll=8)
            def p2(j):
                sl = pl.ds(j, _L)
                fl = idx_v[b][sl]
                g = g_v[b][sl]
                m = jnp.bitwise_and(lax.shift_right_logical(fl, 7), k - 1)
                lo = _take(tab, m)
                sc = _take(tab, m + k)
                g_v[b][sl] = lo + sc * g

        def start_store(c, b):
            off = base + c * chunk
            return pltpu.async_copy(g_v[b], out_hbm.at[pl.ds(off, chunk)], ssem[b])

        gather_d = [None, None]
        store_d = [None, None]
        load_and_index(0, 0)
        gather_d[0] = start_gather(0)
        for c in range(n_ch):
            b = c & 1
            nb = 1 - b
            if c + 1 < n_ch:
                load_and_index(c + 1, nb)
                if store_d[nb] is not None:
                    store_d[nb].wait()
                    store_d[nb] = None
                gather_d[nb] = start_gather(nb)
            gather_d[b].wait()
            transform(b)
            store_d[b] = start_store(c, b)
        for b in range(2):
            if store_d[b] is not None:
                store_d[b].wait()

    return sc_kernel


def kernel(eps, ms, location, scale):
    n, k = eps.shape
    sc_kernel = _build_sc_kernel(n, k)
    eps_native = (
        eps.reshape(n // _LANES, _LANES, k).transpose(0, 2, 1).reshape(n * k)
    )
    tab = jnp.concatenate(
        [location.astype(jnp.float32), scale.astype(jnp.float32)]
    )
    return sc_kernel(eps_native, ms.astype(jnp.int32), tab)


# speedup vs baseline: 19.5020x; 1.9365x over previous
import functools

import jax
import jax.numpy as jnp
from jax import lax
from jax.experimental import pallas as pl
from jax.experimental.pallas import tpu as pltpu
from jax.experimental.pallas import tpu_sc as plsc

_NC = 2
_NS = 16
_NW = _NC * _NS
_L = 16
_LANES = 128

_CHUNK = 4096


def _take(tab, idx):
    return tab.at[idx].get(mode="promise_in_bounds")


@functools.lru_cache(maxsize=None)
def _build_sc_kernel(n: int, k: int):
    assert k == 8, "kernel is specialized to K == 8 mixture components"
    per_w = n // _NW
    assert per_w * _NW == n
    chunk = min(_CHUNK, per_w)
    n_ch = per_w // chunk
    assert n_ch * chunk == per_w
    assert chunk % _LANES == 0 and n % _LANES == 0
    tile = k * _LANES

    mesh = plsc.VectorSubcoreMesh(
        core_axis_name="c", subcore_axis_name="s", num_cores=_NC, num_subcores=_NS
    )

    @functools.partial(
        pl.kernel,
        mesh=mesh,
        compiler_params=pltpu.CompilerParams(needs_layout_passes=False),
        out_type=jax.ShapeDtypeStruct((n,), jnp.float32),
        scratch_types=[
            pltpu.VMEM((chunk * k,), jnp.float32),
            pltpu.VMEM((chunk * k,), jnp.float32),
            pltpu.VMEM((chunk,), jnp.int32),
            pltpu.VMEM((chunk,), jnp.int32),
            pltpu.VMEM((chunk,), jnp.float32),
            pltpu.VMEM((chunk,), jnp.float32),
            pltpu.VMEM((2 * k,), jnp.float32),
            pltpu.SemaphoreType.DMA,
            pltpu.SemaphoreType.DMA,
            pltpu.SemaphoreType.DMA,
            pltpu.SemaphoreType.DMA,
        ],
    )
    def sc_kernel(eps_hbm, ms_hbm, tab_hbm, out_hbm,
                  eb0, eb1, mb0, mb1, ob0, ob1, tab_v, ls0, ls1, ss0, ss1):
        eb = (eb0, eb1)
        mb = (mb0, mb1)
        ob = (ob0, ob1)
        lsem = (ls0, ls1)
        ssem = (ss0, ss1)

        wid = lax.axis_index("s") * _NC + lax.axis_index("c")
        base = wid * per_w

        pltpu.sync_copy(tab_hbm, tab_v)
        tab = tab_v[...]

        iota = lax.iota(jnp.int32, _L)

        def start_loads(c, b):
            off = base + c * chunk
            d1 = pltpu.async_copy(
                eps_hbm.at[pl.ds(off * k, chunk * k)], eb[b], lsem[b]
            )
            d2 = pltpu.async_copy(ms_hbm.at[pl.ds(off, chunk)], mb[b], lsem[b])
            return (d1, d2)

        def compute(b):
            @plsc.parallel_loop(0, chunk, _L, unroll=8)
            def p(j):
                sl = pl.ds(j, _L)
                m = mb[b][sl]
                lo = _take(tab, m)
                sc = _take(tab, m + k)
                s = (j // _LANES) * tile + (j % _LANES)
                lidx = lax.shift_left(m, 7) + (s + iota)
                g = plsc.load_gather(eb[b], [lidx])
                ob[b][sl] = lo + sc * g

        def start_store(c, b):
            off = base + c * chunk
            return pltpu.async_copy(ob[b], out_hbm.at[pl.ds(off, chunk)], ssem[b])

        load_d = [None, None]
        store_d = [None, None]
        load_d[0] = start_loads(0, 0)
        if n_ch > 1:
            load_d[1] = start_loads(1, 1)
        for c in range(n_ch):
            b = c & 1
            for d in load_d[b]:
                d.wait()
            if store_d[b] is not None:
                store_d[b].wait()
                store_d[b] = None
            compute(b)
            store_d[b] = start_store(c, b)
            if c + 2 < n_ch:
                load_d[b] = start_loads(c + 2, b)
        for b in range(2):
            if store_d[b] is not None:
                store_d[b].wait()

    return sc_kernel


def kernel(eps, ms, location, scale):
    n, k = eps.shape
    sc_kernel = _build_sc_kernel(n, k)
    eps_native = (
        eps.reshape(n // _LANES, _LANES, k).transpose(0, 2, 1).reshape(n * k)
    )
    tab = jnp.concatenate(
        [location.astype(jnp.float32), scale.astype(jnp.float32)]
    )
    return sc_kernel(eps_native, ms.astype(jnp.int32), tab)
